# baseline (speedup 1.0000x reference)
import dataclasses
import functools

import jax
import jax.numpy as jnp
from jax import lax
from jax.experimental import pallas as pl
from jax.experimental.pallas import tpu as pltpu
from jax.experimental.pallas import tpu_sc as plsc

N_NODES = 10000
D = 128
DH = 64
NC = 2
NS = 16
NW = NC * NS
L = 16
CHUNK = 128
C0_CHUNKS = 116
C1_CHUNKS = 42
SC1_BASE = NS * C0_CHUNKS
MAX_CHUNKS = C0_CHUNKS
PAD_CHUNKS = SC1_BASE + (NS - 1) * C1_CHUNKS + MAX_CHUNKS
NPAD = 10112
ROWS_PER_TILE = NPAD // NS


def _sc_scatter_pass(feat_l, feat_r, src2d, dst2d, zeros, with_counts):
  mesh = plsc.VectorSubcoreMesh(core_axis_name="c", subcore_axis_name="s")
  out_types = [jax.ShapeDtypeStruct((NC, 2, NPAD, DH), jnp.float32)]
  scratch = [
      pltpu.VMEM((MAX_CHUNKS, CHUNK), jnp.int32),
      pltpu.VMEM((MAX_CHUNKS, CHUNK), jnp.int32),
      pltpu.VMEM((CHUNK, DH), jnp.float32),
      pltpu.VMEM((CHUNK, DH), jnp.float32),
      pltpu.VMEM_SHARED((NPAD, DH), jnp.float32),
      pltpu.SemaphoreType.DMA,
      pltpu.SemaphoreType.DMA,
  ]
  if with_counts:
    out_types.append(jax.ShapeDtypeStruct((NW, NPAD), jnp.float32))
    scratch.append(pltpu.VMEM((NPAD,), jnp.float32))

  cp = pltpu.CompilerParams()
  if "needs_layout_passes" in pltpu.CompilerParams.__dataclass_fields__:
    cp = dataclasses.replace(cp, needs_layout_passes=False)
  if "use_tc_tiling_on_sc" in pltpu.CompilerParams.__dataclass_fields__:
    cp = dataclasses.replace(cp, use_tc_tiling_on_sc=False)

  @functools.partial(
      pl.kernel, out_type=tuple(out_types), mesh=mesh, scratch_types=scratch,
      compiler_params=cp)
  def k(fl_hbm, fr_hbm, src_hbm, dst_hbm, zeros_hbm, out_hbm, *rest):
    if with_counts:
      cnt_hbm, srcv, dstv, buf_a, buf_b, acc, sem_a, sem_b, cntv = rest
    else:
      srcv, dstv, buf_a, buf_b, acc, sem_a, sem_b = rest
    c = lax.axis_index("c")
    s = lax.axis_index("s")
    wid = c * NS + s
    rows = pl.ds(s * ROWS_PER_TILE, ROWS_PER_TILE)
    off = jnp.where(c == 0, s * C0_CHUNKS, SC1_BASE + s * C1_CHUNKS)
    nck = jnp.where(c == 0, C0_CHUNKS, C1_CHUNKS)

    pltpu.sync_copy(src_hbm.at[pl.ds(off, MAX_CHUNKS)], srcv)
    pltpu.sync_copy(dst_hbm.at[pl.ds(off, MAX_CHUNKS)], dstv)

    if with_counts:
      zero16 = jnp.zeros((L,), jnp.float32)

      @pl.loop(0, NPAD // L)
      def _(i):
        cntv[pl.ds(i * L, L)] = zero16

      ones16 = jnp.ones((L,), jnp.float32)

      @pl.loop(0, nck)
      def _(j):
        @pl.loop(0, CHUNK // L)
        def _(q):
          idx = dstv[j, pl.ds(q * L, L)]
          plsc.addupdate_scatter(cntv, [idx], ones16)

      pltpu.sync_copy(cntv, cnt_hbm.at[wid])

    for half, feat_hbm in enumerate((fl_hbm, fr_hbm)):
      pltpu.sync_copy(zeros_hbm.at[rows], acc.at[rows])
      plsc.subcore_barrier()

      pltpu.async_copy(feat_hbm.at[srcv.at[0]], buf_a, sem_a)

      @pl.loop(0, nck, step=2)
      def _(j):
        pltpu.async_copy(feat_hbm.at[srcv.at[j + 1]], buf_b, sem_b)
        pltpu.make_async_copy(feat_hbm.at[srcv.at[j]], buf_a, sem_a).wait()
        pltpu.sync_copy(buf_a, acc.at[dstv.at[j]], add=True)

        @pl.when(j + 2 < nck)
        def _():
          pltpu.async_copy(feat_hbm.at[srcv.at[j + 2]], buf_a, sem_a)

        pltpu.make_async_copy(feat_hbm.at[srcv.at[j + 1]], buf_b, sem_b).wait()
        pltpu.sync_copy(buf_b, acc.at[dstv.at[j + 1]], add=True)

      plsc.subcore_barrier()

      pltpu.sync_copy(acc.at[rows], out_hbm.at[c].at[half].at[rows])
      plsc.subcore_barrier()

  return k(feat_l, feat_r, src2d, dst2d, zeros)


def _dot_t(a, w):
  return lax.dot_general(a, w, (((1,), (1,)), ((), ())),
                         preferred_element_type=jnp.float32)


def _combine(p_ref, c_ref):
  agg = jnp.concatenate(
      [p_ref[0, 0] + p_ref[1, 0], p_ref[0, 1] + p_ref[1, 1]],
      axis=1)[:N_NODES]
  cnt = jnp.sum(c_ref[...], axis=0)[:N_NODES]
  return agg / jnp.maximum(cnt, 1.0)[:, None]


def _tc_layer(parts, cnts, h_in, wl, bl, wr):
  def body(p_ref, c_ref, x_ref, wl_ref, bl_ref, wr_ref, o_l, o_r):
    agg = _combine(p_ref, c_ref)
    h = (_dot_t(agg, wl_ref[...]) + bl_ref[...][None, :]
         + _dot_t(x_ref[...], wr_ref[...]))
    h = jnp.maximum(h, 0.0)
    o_l[...] = h[:, :DH]
    o_r[...] = h[:, DH:]

  return pl.pallas_call(
      body,
      out_shape=(jax.ShapeDtypeStruct((N_NODES, DH), jnp.float32),
                 jax.ShapeDtypeStruct((N_NODES, DH), jnp.float32)),
  )(parts, cnts, h_in, wl, bl, wr)


def _tc_head(parts, cnts, h1l, h1r, w2l, b2l, w2r, wlin1, blin1, wout, bout):
  def body(p_ref, c_ref, h1l_ref, h1r_ref, w2l_ref, b2l_ref, w2r_ref,
           wlin1_ref, blin1_ref, wout_ref, bout_ref, o_ref):
    agg = _combine(p_ref, c_ref)
    h1 = jnp.concatenate([h1l_ref[...], h1r_ref[...]], axis=1)
    h2 = (_dot_t(agg, w2l_ref[...]) + b2l_ref[...][None, :]
          + _dot_t(h1, w2r_ref[...]))
    h2 = jnp.maximum(h2, 0.0)
    g = jnp.mean(h2, axis=0, keepdims=True)
    z = _dot_t(g, wlin1_ref[...]) + blin1_ref[...][None, :]
    z = jnp.maximum(z, 0.0)
    ssum = jnp.sum(z * wout_ref[...]) + bout_ref[0]
    sv = jax.nn.sigmoid(ssum)
    o_ref[...] = jnp.full((N_NODES, 1), sv, jnp.float32)

  return pl.pallas_call(
      body,
      out_shape=jax.ShapeDtypeStruct((N_NODES, 1), jnp.float32),
  )(parts, cnts, h1l, h1r, w2l, b2l, w2r, wlin1, blin1, wout, bout)


@jax.jit
def kernel(x, edge_index, W1l, b1l, W1r, W2l, b2l, W2r, Wlin1, blin1, Wout,
           bout):
  e = edge_index.astype(jnp.int32)
  src, dst = e[0], e[1]
  epad = PAD_CHUNKS * CHUNK
  n_extra = epad - src.shape[0]
  pad_dst = N_NODES + jnp.arange(n_extra, dtype=jnp.int32) % (NPAD - N_NODES)
  src2d = jnp.concatenate(
      [src, jnp.zeros((n_extra,), jnp.int32)]).reshape(PAD_CHUNKS, CHUNK)
  dst2d = jnp.concatenate([dst, pad_dst]).reshape(PAD_CHUNKS, CHUNK)
  zeros = jnp.zeros((NPAD, DH), jnp.float32)

  parts1, cnts = _sc_scatter_pass(
      x[:, :DH], x[:, DH:], src2d, dst2d, zeros, with_counts=True)
  h1l, h1r = _tc_layer(parts1, cnts, x, W1l, b1l, W1r)
  (parts2,) = _sc_scatter_pass(
      h1l, h1r, src2d, dst2d, zeros, with_counts=False)
  return _tc_head(parts2, cnts, h1l, h1r, W2l, b2l, W2r, Wlin1, blin1, Wout,
                  bout)


# speedup vs baseline: 1.0302x; 1.0302x over previous
import dataclasses
import functools

import jax
import jax.numpy as jnp
from jax import lax
from jax.experimental import pallas as pl
from jax.experimental.pallas import tpu as pltpu
from jax.experimental.pallas import tpu_sc as plsc

N_NODES = 10000
D = 128
DH = 64
NC = 2
NS = 16
NW = NC * NS
L = 16
CHUNK = 128
C0_CHUNKS = 118
C1_CHUNKS = 40
SC1_BASE = NS * C0_CHUNKS
MAX_CHUNKS = C0_CHUNKS
PAD_CHUNKS = SC1_BASE + (NS - 1) * C1_CHUNKS + MAX_CHUNKS
NPAD = 10112
ROWS_PER_TILE = NPAD // NS


def _sc_scatter_pass(feat_l, feat_r, src2d, dst2d, zeros, with_counts):
  mesh = plsc.VectorSubcoreMesh(core_axis_name="c", subcore_axis_name="s")
  out_types = [jax.ShapeDtypeStruct((NC, 2, NPAD, DH), jnp.float32)]
  scratch = [
      pltpu.VMEM((MAX_CHUNKS, CHUNK), jnp.int32),
      pltpu.VMEM((MAX_CHUNKS, CHUNK), jnp.int32),
      pltpu.VMEM((CHUNK, DH), jnp.float32),
      pltpu.VMEM((CHUNK, DH), jnp.float32),
      pltpu.VMEM_SHARED((NPAD, DH), jnp.float32),
      pltpu.SemaphoreType.DMA,
      pltpu.SemaphoreType.DMA,
  ]
  if with_counts:
    out_types.append(jax.ShapeDtypeStruct((NW, NPAD), jnp.float32))
    scratch.append(pltpu.VMEM((NPAD,), jnp.float32))

  cp = pltpu.CompilerParams()
  if "needs_layout_passes" in pltpu.CompilerParams.__dataclass_fields__:
    cp = dataclasses.replace(cp, needs_layout_passes=False)
  if "use_tc_tiling_on_sc" in pltpu.CompilerParams.__dataclass_fields__:
    cp = dataclasses.replace(cp, use_tc_tiling_on_sc=False)

  @functools.partial(
      pl.kernel, out_type=tuple(out_types), mesh=mesh, scratch_types=scratch,
      compiler_params=cp)
  def k(fl_hbm, fr_hbm, src_hbm, dst_hbm, zeros_hbm, out_hbm, *rest):
    if with_counts:
      cnt_hbm, srcv, dstv, buf_a, buf_b, acc, sem_a, sem_b, cntv = rest
    else:
      srcv, dstv, buf_a, buf_b, acc, sem_a, sem_b = rest
    c = lax.axis_index("c")
    s = lax.axis_index("s")
    wid = c * NS + s
    rows = pl.ds(s * ROWS_PER_TILE, ROWS_PER_TILE)
    off = jnp.where(c == 0, s * C0_CHUNKS, SC1_BASE + s * C1_CHUNKS)
    nck = jnp.where(c == 0, C0_CHUNKS, C1_CHUNKS)

    pltpu.sync_copy(src_hbm.at[pl.ds(off, MAX_CHUNKS)], srcv)
    pltpu.sync_copy(dst_hbm.at[pl.ds(off, MAX_CHUNKS)], dstv)

    if with_counts:
      zero16 = jnp.zeros((L,), jnp.float32)

      @pl.loop(0, NPAD // L)
      def _(i):
        cntv[pl.ds(i * L, L)] = zero16

      ones16 = jnp.ones((L,), jnp.float32)

      @pl.loop(0, nck)
      def _(j):
        @pl.loop(0, CHUNK // L)
        def _(q):
          idx = dstv[j, pl.ds(q * L, L)]
          plsc.addupdate_scatter(cntv, [idx], ones16)

      pltpu.sync_copy(cntv, cnt_hbm.at[wid])

    for half, feat_hbm in enumerate((fl_hbm, fr_hbm)):
      pltpu.sync_copy(zeros_hbm.at[rows], acc.at[rows])
      plsc.subcore_barrier()

      pltpu.async_copy(feat_hbm.at[srcv.at[0]], buf_a, sem_a)

      @pl.loop(0, nck, step=2)
      def _(j):
        pltpu.async_copy(feat_hbm.at[srcv.at[j + 1]], buf_b, sem_b)
        pltpu.make_async_copy(feat_hbm.at[srcv.at[j]], buf_a, sem_a).wait()
        pltpu.sync_copy(buf_a, acc.at[dstv.at[j]], add=True)

        @pl.when(j + 2 < nck)
        def _():
          pltpu.async_copy(feat_hbm.at[srcv.at[j + 2]], buf_a, sem_a)

        pltpu.make_async_copy(feat_hbm.at[srcv.at[j + 1]], buf_b, sem_b).wait()
        pltpu.sync_copy(buf_b, acc.at[dstv.at[j + 1]], add=True)

      plsc.subcore_barrier()

      pltpu.sync_copy(acc.at[rows], out_hbm.at[c].at[half].at[rows])
      plsc.subcore_barrier()

  return k(feat_l, feat_r, src2d, dst2d, zeros)


def _dot_t(a, w):
  return lax.dot_general(a, w, (((1,), (1,)), ((), ())),
                         preferred_element_type=jnp.float32)


def _combine(p_ref, c_ref):
  agg = jnp.concatenate(
      [p_ref[0, 0] + p_ref[1, 0], p_ref[0, 1] + p_ref[1, 1]],
      axis=1)[:N_NODES]
  cnt = jnp.sum(c_ref[...], axis=0)[:N_NODES]
  return agg / jnp.maximum(cnt, 1.0)[:, None]


def _tc_layer(parts, cnts, h_in, wl, bl, wr):
  def body(p_ref, c_ref, x_ref, wl_ref, bl_ref, wr_ref, o_ref):
    agg = _combine(p_ref, c_ref)
    h = (_dot_t(agg, wl_ref[...]) + bl_ref[...][None, :]
         + _dot_t(x_ref[...], wr_ref[...]))
    o_ref[...] = jnp.maximum(h, 0.0)

  return pl.pallas_call(
      body,
      out_shape=jax.ShapeDtypeStruct((N_NODES, D), jnp.float32),
  )(parts, cnts, h_in, wl, bl, wr)


def _tc_head(parts, cnts, h1, w2l, b2l, w2r, wlin1, blin1, wout, bout):
  def body(p_ref, c_ref, h1_ref, w2l_ref, b2l_ref, w2r_ref, wlin1_ref,
           blin1_ref, wout_ref, bout_ref, o_ref):
    agg = _combine(p_ref, c_ref)
    h2 = (_dot_t(agg, w2l_ref[...]) + b2l_ref[...][None, :]
          + _dot_t(h1_ref[...], w2r_ref[...]))
    h2 = jnp.maximum(h2, 0.0)
    g = jnp.mean(h2, axis=0, keepdims=True)
    z = _dot_t(g, wlin1_ref[...]) + blin1_ref[...][None, :]
    z = jnp.maximum(z, 0.0)
    ssum = jnp.sum(z * wout_ref[...]) + bout_ref[0]
    sv = jax.nn.sigmoid(ssum)
    o_ref[...] = jnp.full((N_NODES, 1), sv, jnp.float32)

  return pl.pallas_call(
      body,
      out_shape=jax.ShapeDtypeStruct((N_NODES, 1), jnp.float32),
  )(parts, cnts, h1, w2l, b2l, w2r, wlin1, blin1, wout, bout)


@jax.jit
def kernel(x, edge_index, W1l, b1l, W1r, W2l, b2l, W2r, Wlin1, blin1, Wout,
           bout):
  e = edge_index.astype(jnp.int32)
  src, dst = e[0], e[1]
  epad = PAD_CHUNKS * CHUNK
  n_extra = epad - src.shape[0]
  pad_dst = N_NODES + jnp.arange(n_extra, dtype=jnp.int32) % (NPAD - N_NODES)
  src2d = jnp.concatenate(
      [src, jnp.zeros((n_extra,), jnp.int32)]).reshape(PAD_CHUNKS, CHUNK)
  dst2d = jnp.concatenate([dst, pad_dst]).reshape(PAD_CHUNKS, CHUNK)
  zeros = jnp.zeros((NPAD, DH), jnp.float32)

  parts1, cnts = _sc_scatter_pass(
      x[:, :DH], x[:, DH:], src2d, dst2d, zeros, with_counts=True)
  h1 = _tc_layer(parts1, cnts, x, W1l, b1l, W1r)
  (parts2,) = _sc_scatter_pass(
      h1[:, :DH], h1[:, DH:], src2d, dst2d, zeros, with_counts=False)
  return _tc_head(parts2, cnts, h1, W2l, b2l, W2r, Wlin1, blin1, Wout, bout)


# speedup vs baseline: 1.0384x; 1.0080x over previous
import dataclasses
import functools

import jax
import jax.numpy as jnp
from jax import lax
from jax.experimental import pallas as pl
from jax.experimental.pallas import tpu as pltpu
from jax.experimental.pallas import tpu_sc as plsc

N_NODES = 10000
D = 128
DH = 64
NC = 2
NS = 16
NW = NC * NS
L = 16
CHUNK = 128
C0_CHUNKS = 118
C1_CHUNKS = 40
SC1_BASE = NS * C0_CHUNKS
MAX_CHUNKS = C0_CHUNKS
PAD_CHUNKS = SC1_BASE + (NS - 1) * C1_CHUNKS + MAX_CHUNKS
NPAD = 10112
ROWS_PER_TILE = NPAD // NS


def _sc_scatter_pass(feat_l, feat_r, src2d, dst2d, with_counts):
  mesh = plsc.VectorSubcoreMesh(core_axis_name="c", subcore_axis_name="s")
  out_types = [jax.ShapeDtypeStruct((NC, 2, NPAD, DH), jnp.float32)]
  scratch = [
      pltpu.VMEM((MAX_CHUNKS, CHUNK), jnp.int32),
      pltpu.VMEM((MAX_CHUNKS, CHUNK), jnp.int32),
      pltpu.VMEM((CHUNK, DH), jnp.float32),
      pltpu.VMEM((CHUNK, DH), jnp.float32),
      pltpu.VMEM((CHUNK, DH), jnp.float32),
      pltpu.VMEM_SHARED((NPAD, DH), jnp.float32),
      pltpu.SemaphoreType.DMA,
      pltpu.SemaphoreType.DMA,
  ]
  if with_counts:
    out_types.append(jax.ShapeDtypeStruct((NW, NPAD), jnp.float32))
    scratch.append(pltpu.VMEM((NPAD,), jnp.float32))

  cp = pltpu.CompilerParams()
  if "needs_layout_passes" in pltpu.CompilerParams.__dataclass_fields__:
    cp = dataclasses.replace(cp, needs_layout_passes=False)
  if "use_tc_tiling_on_sc" in pltpu.CompilerParams.__dataclass_fields__:
    cp = dataclasses.replace(cp, use_tc_tiling_on_sc=False)

  @functools.partial(
      pl.kernel, out_type=tuple(out_types), mesh=mesh, scratch_types=scratch,
      compiler_params=cp)
  def k(fl_hbm, fr_hbm, src_hbm, dst_hbm, out_hbm, *rest):
    if with_counts:
      cnt_hbm, srcv, dstv, buf_a, buf_b, zbuf, acc, sem_a, sem_b, cntv = rest
    else:
      srcv, dstv, buf_a, buf_b, zbuf, acc, sem_a, sem_b = rest
    c = lax.axis_index("c")
    s = lax.axis_index("s")
    wid = c * NS + s
    rows = pl.ds(s * ROWS_PER_TILE, ROWS_PER_TILE)
    off = jnp.where(c == 0, s * C0_CHUNKS, SC1_BASE + s * C1_CHUNKS)
    nck = jnp.where(c == 0, C0_CHUNKS, C1_CHUNKS)

    pltpu.sync_copy(src_hbm.at[pl.ds(off, MAX_CHUNKS)], srcv)
    pltpu.sync_copy(dst_hbm.at[pl.ds(off, MAX_CHUNKS)], dstv)

    zero16 = jnp.zeros((L,), jnp.float32)

    @pl.loop(0, CHUNK)
    def _(r):
      for q in range(DH // L):
        zbuf[r, pl.ds(q * L, L)] = zero16

    if with_counts:
      @pl.loop(0, NPAD // L)
      def _(i):
        cntv[pl.ds(i * L, L)] = zero16

      ones16 = jnp.ones((L,), jnp.float32)

      @pl.loop(0, nck)
      def _(j):
        @pl.loop(0, CHUNK // L)
        def _(q):
          idx = dstv[j, pl.ds(q * L, L)]
          plsc.addupdate_scatter(cntv, [idx], ones16)

      pltpu.sync_copy(cntv, cnt_hbm.at[wid])

    for half, feat_hbm in enumerate((fl_hbm, fr_hbm)):
      base = s * ROWS_PER_TILE
      for kk in range(ROWS_PER_TILE // CHUNK):
        pltpu.sync_copy(zbuf, acc.at[pl.ds(base + kk * CHUNK, CHUNK)])
      rem = ROWS_PER_TILE % CHUNK
      if rem:
        pltpu.sync_copy(
            zbuf.at[pl.ds(0, rem)],
            acc.at[pl.ds(base + ROWS_PER_TILE - rem, rem)])
      plsc.subcore_barrier()

      pltpu.async_copy(feat_hbm.at[srcv.at[0]], buf_a, sem_a)

      @pl.loop(0, nck, step=2)
      def _(j):
        pltpu.async_copy(feat_hbm.at[srcv.at[j + 1]], buf_b, sem_b)
        pltpu.make_async_copy(feat_hbm.at[srcv.at[j]], buf_a, sem_a).wait()
        pltpu.sync_copy(buf_a, acc.at[dstv.at[j]], add=True)

        @pl.when(j + 2 < nck)
        def _():
          pltpu.async_copy(feat_hbm.at[srcv.at[j + 2]], buf_a, sem_a)

        pltpu.make_async_copy(feat_hbm.at[srcv.at[j + 1]], buf_b, sem_b).wait()
        pltpu.sync_copy(buf_b, acc.at[dstv.at[j + 1]], add=True)

      plsc.subcore_barrier()

      pltpu.sync_copy(acc.at[rows], out_hbm.at[c].at[half].at[rows])
      plsc.subcore_barrier()

  return k(feat_l, feat_r, src2d, dst2d)


def _dot_t(a, w):
  return lax.dot_general(a, w, (((1,), (1,)), ((), ())),
                         preferred_element_type=jnp.float32)


def _combine(p_ref, c_ref):
  agg = jnp.concatenate(
      [p_ref[0, 0] + p_ref[1, 0], p_ref[0, 1] + p_ref[1, 1]],
      axis=1)[:N_NODES]
  cnt = jnp.sum(c_ref[...], axis=0)[:N_NODES]
  return agg / jnp.maximum(cnt, 1.0)[:, None]


def _tc_layer(parts, cnts, h_in, wl, bl, wr):
  def body(p_ref, c_ref, x_ref, wl_ref, bl_ref, wr_ref, o_ref):
    agg = _combine(p_ref, c_ref)
    h = (_dot_t(agg, wl_ref[...]) + bl_ref[...][None, :]
         + _dot_t(x_ref[...], wr_ref[...]))
    o_ref[...] = jnp.maximum(h, 0.0)

  return pl.pallas_call(
      body,
      out_shape=jax.ShapeDtypeStruct((N_NODES, D), jnp.float32),
  )(parts, cnts, h_in, wl, bl, wr)


def _tc_head(parts, cnts, h1, w2l, b2l, w2r, wlin1, blin1, wout, bout):
  def body(p_ref, c_ref, h1_ref, w2l_ref, b2l_ref, w2r_ref, wlin1_ref,
           blin1_ref, wout_ref, bout_ref, o_ref):
    agg = _combine(p_ref, c_ref)
    h2 = (_dot_t(agg, w2l_ref[...]) + b2l_ref[...][None, :]
          + _dot_t(h1_ref[...], w2r_ref[...]))
    h2 = jnp.maximum(h2, 0.0)
    g = jnp.mean(h2, axis=0, keepdims=True)
    z = _dot_t(g, wlin1_ref[...]) + blin1_ref[...][None, :]
    z = jnp.maximum(z, 0.0)
    ssum = jnp.sum(z * wout_ref[...]) + bout_ref[0]
    sv = jax.nn.sigmoid(ssum)
    o_ref[...] = jnp.full((N_NODES, 1), sv, jnp.float32)

  return pl.pallas_call(
      body,
      out_shape=jax.ShapeDtypeStruct((N_NODES, 1), jnp.float32),
  )(parts, cnts, h1, w2l, b2l, w2r, wlin1, blin1, wout, bout)


@jax.jit
def kernel(x, edge_index, W1l, b1l, W1r, W2l, b2l, W2r, Wlin1, blin1, Wout,
           bout):
  e = edge_index.astype(jnp.int32)
  src, dst = e[0], e[1]
  epad = PAD_CHUNKS * CHUNK
  n_extra = epad - src.shape[0]
  pad_dst = N_NODES + jnp.arange(n_extra, dtype=jnp.int32) % (NPAD - N_NODES)
  src2d = jnp.concatenate(
      [src, jnp.zeros((n_extra,), jnp.int32)]).reshape(PAD_CHUNKS, CHUNK)
  dst2d = jnp.concatenate([dst, pad_dst]).reshape(PAD_CHUNKS, CHUNK)

  parts1, cnts = _sc_scatter_pass(
      x[:, :DH], x[:, DH:], src2d, dst2d, with_counts=True)
  h1 = _tc_layer(parts1, cnts, x, W1l, b1l, W1r)
  (parts2,) = _sc_scatter_pass(
      h1[:, :DH], h1[:, DH:], src2d, dst2d, with_counts=False)
  return _tc_head(parts2, cnts, h1, W2l, b2l, W2r, Wlin1, blin1, Wout, bout)


# speedup vs baseline: 1.0866x; 1.0464x over previous
import dataclasses
import functools

import jax
import jax.numpy as jnp
from jax import lax
from jax.experimental import pallas as pl
from jax.experimental.pallas import tpu as pltpu
from jax.experimental.pallas import tpu_sc as plsc

N_NODES = 10000
D = 128
DH = 64
NC = 2
NS = 16
NW = NC * NS
L = 16
CHUNK = 128
C0_CHUNKS = 118
C1_CHUNKS = 40
SC1_BASE = NS * C0_CHUNKS
MAX_CHUNKS = C0_CHUNKS
PAD_CHUNKS = SC1_BASE + (NS - 1) * C1_CHUNKS + MAX_CHUNKS
NPAD = 10112
ROWS_PER_TILE = NPAD // NS


def _sc_scatter_pass(feat_l, feat_r, src2d, dst2d, with_counts):
  mesh = plsc.VectorSubcoreMesh(core_axis_name="c", subcore_axis_name="s")
  out_types = [jax.ShapeDtypeStruct((NC, NPAD, D), jnp.float32)]
  scratch = [
      pltpu.VMEM((MAX_CHUNKS, CHUNK), jnp.int32),
      pltpu.VMEM((MAX_CHUNKS, CHUNK), jnp.int32),
      pltpu.VMEM((CHUNK, DH), jnp.float32),
      pltpu.VMEM((CHUNK, DH), jnp.float32),
      pltpu.VMEM((CHUNK, DH), jnp.float32),
      pltpu.VMEM_SHARED((NPAD, DH), jnp.float32),
      pltpu.SemaphoreType.DMA,
      pltpu.SemaphoreType.DMA,
  ]
  if with_counts:
    out_types.append(jax.ShapeDtypeStruct((NW, NPAD), jnp.float32))
    scratch.append(pltpu.VMEM((NPAD,), jnp.float32))

  cp = pltpu.CompilerParams()
  if "needs_layout_passes" in pltpu.CompilerParams.__dataclass_fields__:
    cp = dataclasses.replace(cp, needs_layout_passes=False)
  if "use_tc_tiling_on_sc" in pltpu.CompilerParams.__dataclass_fields__:
    cp = dataclasses.replace(cp, use_tc_tiling_on_sc=False)

  @functools.partial(
      pl.kernel, out_type=tuple(out_types), mesh=mesh, scratch_types=scratch,
      compiler_params=cp)
  def k(fl_hbm, fr_hbm, src_hbm, dst_hbm, out_hbm, *rest):
    if with_counts:
      cnt_hbm, srcv, dstv, buf_a, buf_b, zbuf, acc, sem_a, sem_b, cntv = rest
    else:
      srcv, dstv, buf_a, buf_b, zbuf, acc, sem_a, sem_b = rest
    c = lax.axis_index("c")
    s = lax.axis_index("s")
    wid = c * NS + s
    rows = pl.ds(s * ROWS_PER_TILE, ROWS_PER_TILE)
    off = jnp.where(c == 0, s * C0_CHUNKS, SC1_BASE + s * C1_CHUNKS)
    nck = jnp.where(c == 0, C0_CHUNKS, C1_CHUNKS)

    pltpu.sync_copy(src_hbm.at[pl.ds(off, MAX_CHUNKS)], srcv)
    pltpu.sync_copy(dst_hbm.at[pl.ds(off, MAX_CHUNKS)], dstv)

    zero16 = jnp.zeros((L,), jnp.float32)

    @pl.loop(0, CHUNK)
    def _(r):
      for q in range(DH // L):
        zbuf[r, pl.ds(q * L, L)] = zero16

    if with_counts:
      @pl.loop(0, NPAD // L)
      def _(i):
        cntv[pl.ds(i * L, L)] = zero16

      ones16 = jnp.ones((L,), jnp.float32)

      @pl.loop(0, nck)
      def _(j):
        @pl.loop(0, CHUNK // L)
        def _(q):
          idx = dstv[j, pl.ds(q * L, L)]
          plsc.addupdate_scatter(cntv, [idx], ones16)

      pltpu.sync_copy(cntv, cnt_hbm.at[wid])

    for half, feat_hbm in enumerate((fl_hbm, fr_hbm)):
      base = s * ROWS_PER_TILE
      for kk in range(ROWS_PER_TILE // CHUNK):
        pltpu.sync_copy(zbuf, acc.at[pl.ds(base + kk * CHUNK, CHUNK)])
      rem = ROWS_PER_TILE % CHUNK
      if rem:
        pltpu.sync_copy(
            zbuf.at[pl.ds(0, rem)],
            acc.at[pl.ds(base + ROWS_PER_TILE - rem, rem)])
      plsc.subcore_barrier()

      pltpu.async_copy(feat_hbm.at[srcv.at[0]], buf_a, sem_a)

      @pl.loop(0, nck, step=2)
      def _(j):
        pltpu.async_copy(feat_hbm.at[srcv.at[j + 1]], buf_b, sem_b)
        pltpu.make_async_copy(feat_hbm.at[srcv.at[j]], buf_a, sem_a).wait()
        pltpu.sync_copy(buf_a, acc.at[dstv.at[j]], add=True)

        @pl.when(j + 2 < nck)
        def _():
          pltpu.async_copy(feat_hbm.at[srcv.at[j + 2]], buf_a, sem_a)

        pltpu.make_async_copy(feat_hbm.at[srcv.at[j + 1]], buf_b, sem_b).wait()
        pltpu.sync_copy(buf_b, acc.at[dstv.at[j + 1]], add=True)

      plsc.subcore_barrier()

      pltpu.sync_copy(acc.at[rows],
                      out_hbm.at[c].at[rows, pl.ds(half * DH, DH)])
      plsc.subcore_barrier()

  return k(feat_l, feat_r, src2d, dst2d)


def _dot_t(a, w):
  return lax.dot_general(a, w, (((1,), (1,)), ((), ())),
                         preferred_element_type=jnp.float32)


def _combine(p_ref, c_ref):
  agg = (p_ref[0] + p_ref[1])[:N_NODES]
  cnt = jnp.sum(c_ref[...], axis=0)[:N_NODES]
  return agg / jnp.maximum(cnt, 1.0)[:, None]


def _tc_layer(parts, cnts, h_in, wl, bl, wr):
  def body(p_ref, c_ref, x_ref, wl_ref, bl_ref, wr_ref, o_ref):
    agg = _combine(p_ref, c_ref)
    h = (_dot_t(agg, wl_ref[...]) + bl_ref[...][None, :]
         + _dot_t(x_ref[...], wr_ref[...]))
    o_ref[...] = jnp.maximum(h, 0.0)

  return pl.pallas_call(
      body,
      out_shape=jax.ShapeDtypeStruct((N_NODES, D), jnp.float32),
  )(parts, cnts, h_in, wl, bl, wr)


def _tc_head(parts, cnts, h1, w2l, b2l, w2r, wlin1, blin1, wout, bout):
  def body(p_ref, c_ref, h1_ref, w2l_ref, b2l_ref, w2r_ref, wlin1_ref,
           blin1_ref, wout_ref, bout_ref, o_ref):
    agg = _combine(p_ref, c_ref)
    h2 = (_dot_t(agg, w2l_ref[...]) + b2l_ref[...][None, :]
          + _dot_t(h1_ref[...], w2r_ref[...]))
    h2 = jnp.maximum(h2, 0.0)
    g = jnp.mean(h2, axis=0, keepdims=True)
    z = _dot_t(g, wlin1_ref[...]) + blin1_ref[...][None, :]
    z = jnp.maximum(z, 0.0)
    ssum = jnp.sum(z * wout_ref[...]) + bout_ref[0]
    sv = jax.nn.sigmoid(ssum)
    o_ref[...] = jnp.full((N_NODES, 1), sv, jnp.float32)

  return pl.pallas_call(
      body,
      out_shape=jax.ShapeDtypeStruct((N_NODES, 1), jnp.float32),
  )(parts, cnts, h1, w2l, b2l, w2r, wlin1, blin1, wout, bout)


@jax.jit
def kernel(x, edge_index, W1l, b1l, W1r, W2l, b2l, W2r, Wlin1, blin1, Wout,
           bout):
  e = edge_index.astype(jnp.int32)
  src, dst = e[0], e[1]
  epad = PAD_CHUNKS * CHUNK
  n_extra = epad - src.shape[0]
  pad_dst = N_NODES + jnp.arange(n_extra, dtype=jnp.int32) % (NPAD - N_NODES)
  src2d = jnp.concatenate(
      [src, jnp.zeros((n_extra,), jnp.int32)]).reshape(PAD_CHUNKS, CHUNK)
  dst2d = jnp.concatenate([dst, pad_dst]).reshape(PAD_CHUNKS, CHUNK)

  parts1, cnts = _sc_scatter_pass(
      x[:, :DH], x[:, DH:], src2d, dst2d, with_counts=True)
  h1 = _tc_layer(parts1, cnts, x, W1l, b1l, W1r)
  (parts2,) = _sc_scatter_pass(
      h1[:, :DH], h1[:, DH:], src2d, dst2d, with_counts=False)
  return _tc_head(parts2, cnts, h1, W2l, b2l, W2r, Wlin1, blin1, Wout, bout)


# speedup vs baseline: 1.0953x; 1.0081x over previous
import dataclasses
import functools

import jax
import jax.numpy as jnp
from jax import lax
from jax.experimental import pallas as pl
from jax.experimental.pallas import tpu as pltpu
from jax.experimental.pallas import tpu_sc as plsc

N_NODES = 10000
D = 128
DH = 64
NC = 2
NS = 16
NW = NC * NS
L = 16
CHUNK = 128
C0_CHUNKS = 122
C1_CHUNKS = 36
SC1_BASE = NS * C0_CHUNKS
MAX_CHUNKS = C0_CHUNKS
PAD_CHUNKS = SC1_BASE + (NS - 1) * C1_CHUNKS + MAX_CHUNKS
NPAD = 10112
ROWS_PER_TILE = NPAD // NS


def _sc_scatter_pass(feat_l, feat_r, src2d, dst2d, with_counts):
  mesh = plsc.VectorSubcoreMesh(core_axis_name="c", subcore_axis_name="s")
  out_types = [jax.ShapeDtypeStruct((NC, NPAD, D), jnp.float32)]
  scratch = [
      pltpu.VMEM((MAX_CHUNKS, CHUNK), jnp.int32),
      pltpu.VMEM((MAX_CHUNKS, CHUNK), jnp.int32),
      pltpu.VMEM((CHUNK, DH), jnp.float32),
      pltpu.VMEM((CHUNK, DH), jnp.float32),
      pltpu.VMEM((CHUNK, DH), jnp.float32),
      pltpu.VMEM_SHARED((NPAD, DH), jnp.float32),
      pltpu.SemaphoreType.DMA,
      pltpu.SemaphoreType.DMA,
  ]
  if with_counts:
    out_types.append(jax.ShapeDtypeStruct((NW, NPAD), jnp.float32))
    scratch.append(pltpu.VMEM((NPAD,), jnp.float32))

  cp = pltpu.CompilerParams()
  if "needs_layout_passes" in pltpu.CompilerParams.__dataclass_fields__:
    cp = dataclasses.replace(cp, needs_layout_passes=False)
  if "use_tc_tiling_on_sc" in pltpu.CompilerParams.__dataclass_fields__:
    cp = dataclasses.replace(cp, use_tc_tiling_on_sc=False)

  @functools.partial(
      pl.kernel, out_type=tuple(out_types), mesh=mesh, scratch_types=scratch,
      compiler_params=cp)
  def k(fl_hbm, fr_hbm, src_hbm, dst_hbm, out_hbm, *rest):
    if with_counts:
      cnt_hbm, srcv, dstv, buf_a, buf_b, zbuf, acc, sem_a, sem_b, cntv = rest
    else:
      srcv, dstv, buf_a, buf_b, zbuf, acc, sem_a, sem_b = rest
    c = lax.axis_index("c")
    s = lax.axis_index("s")
    wid = c * NS + s
    rows = pl.ds(s * ROWS_PER_TILE, ROWS_PER_TILE)
    off = jnp.where(c == 0, s * C0_CHUNKS, SC1_BASE + s * C1_CHUNKS)
    nck = jnp.where(c == 0, C0_CHUNKS, C1_CHUNKS)

    pltpu.sync_copy(src_hbm.at[pl.ds(off, MAX_CHUNKS)], srcv)
    pltpu.sync_copy(dst_hbm.at[pl.ds(off, MAX_CHUNKS)], dstv)

    zero16 = jnp.zeros((L,), jnp.float32)

    @pl.loop(0, CHUNK)
    def _(r):
      for q in range(DH // L):
        zbuf[r, pl.ds(q * L, L)] = zero16

    if with_counts:
      @pl.loop(0, NPAD // L)
      def _(i):
        cntv[pl.ds(i * L, L)] = zero16

    ones16 = jnp.ones((L,), jnp.float32)

    for half, feat_hbm in enumerate((fl_hbm, fr_hbm)):
      base = s * ROWS_PER_TILE
      for kk in range(ROWS_PER_TILE // CHUNK):
        pltpu.sync_copy(zbuf, acc.at[pl.ds(base + kk * CHUNK, CHUNK)])
      rem = ROWS_PER_TILE % CHUNK
      if rem:
        pltpu.sync_copy(
            zbuf.at[pl.ds(0, rem)],
            acc.at[pl.ds(base + ROWS_PER_TILE - rem, rem)])
      plsc.subcore_barrier()

      pltpu.async_copy(feat_hbm.at[srcv.at[0]], buf_a, sem_a)

      do_counts = with_counts and half == 0

      @pl.loop(0, nck, step=2)
      def _(j):
        pltpu.async_copy(feat_hbm.at[srcv.at[j + 1]], buf_b, sem_b)
        if do_counts:
          for jo in (0, 1):
            for q in range(CHUNK // L):
              idx = dstv[j + jo, pl.ds(q * L, L)]
              plsc.addupdate_scatter(cntv, [idx], ones16)
        pltpu.make_async_copy(feat_hbm.at[srcv.at[j]], buf_a, sem_a).wait()
        pltpu.sync_copy(buf_a, acc.at[dstv.at[j]], add=True)

        @pl.when(j + 2 < nck)
        def _():
          pltpu.async_copy(feat_hbm.at[srcv.at[j + 2]], buf_a, sem_a)

        pltpu.make_async_copy(feat_hbm.at[srcv.at[j + 1]], buf_b, sem_b).wait()
        pltpu.sync_copy(buf_b, acc.at[dstv.at[j + 1]], add=True)

      if do_counts:
        pltpu.sync_copy(cntv, cnt_hbm.at[wid])
      plsc.subcore_barrier()

      pltpu.sync_copy(acc.at[rows],
                      out_hbm.at[c].at[rows, pl.ds(half * DH, DH)])
      plsc.subcore_barrier()

  return k(feat_l, feat_r, src2d, dst2d)


def _dot_t(a, w):
  return lax.dot_general(a, w, (((1,), (1,)), ((), ())),
                         preferred_element_type=jnp.float32)


def _combine(p_ref, c_ref):
  agg = (p_ref[0] + p_ref[1])[:N_NODES]
  cnt = jnp.sum(c_ref[...], axis=0)[:N_NODES]
  return agg / jnp.maximum(cnt, 1.0)[:, None]


def _tc_layer(parts, cnts, h_in, wl, bl, wr):
  def body(p_ref, c_ref, x_ref, wl_ref, bl_ref, wr_ref, o_ref):
    agg = _combine(p_ref, c_ref)
    h = (_dot_t(agg, wl_ref[...]) + bl_ref[...][None, :]
         + _dot_t(x_ref[...], wr_ref[...]))
    o_ref[...] = jnp.maximum(h, 0.0)

  return pl.pallas_call(
      body,
      out_shape=jax.ShapeDtypeStruct((N_NODES, D), jnp.float32),
  )(parts, cnts, h_in, wl, bl, wr)


def _tc_head(parts, cnts, h1, w2l, b2l, w2r, wlin1, blin1, wout, bout):
  def body(p_ref, c_ref, h1_ref, w2l_ref, b2l_ref, w2r_ref, wlin1_ref,
           blin1_ref, wout_ref, bout_ref, o_ref):
    agg = _combine(p_ref, c_ref)
    h2 = (_dot_t(agg, w2l_ref[...]) + b2l_ref[...][None, :]
          + _dot_t(h1_ref[...], w2r_ref[...]))
    h2 = jnp.maximum(h2, 0.0)
    g = jnp.mean(h2, axis=0, keepdims=True)
    z = _dot_t(g, wlin1_ref[...]) + blin1_ref[...][None, :]
    z = jnp.maximum(z, 0.0)
    ssum = jnp.sum(z * wout_ref[...]) + bout_ref[0]
    sv = jax.nn.sigmoid(ssum)
    o_ref[...] = jnp.full((N_NODES, 1), sv, jnp.float32)

  return pl.pallas_call(
      body,
      out_shape=jax.ShapeDtypeStruct((N_NODES, 1), jnp.float32),
  )(parts, cnts, h1, w2l, b2l, w2r, wlin1, blin1, wout, bout)


@jax.jit
def kernel(x, edge_index, W1l, b1l, W1r, W2l, b2l, W2r, Wlin1, blin1, Wout,
           bout):
  e = edge_index.astype(jnp.int32)
  src, dst = e[0], e[1]
  epad = PAD_CHUNKS * CHUNK
  n_extra = epad - src.shape[0]
  pad_dst = N_NODES + jnp.arange(n_extra, dtype=jnp.int32) % (NPAD - N_NODES)
  src2d = jnp.concatenate(
      [src, jnp.zeros((n_extra,), jnp.int32)]).reshape(PAD_CHUNKS, CHUNK)
  dst2d = jnp.concatenate([dst, pad_dst]).reshape(PAD_CHUNKS, CHUNK)

  parts1, cnts = _sc_scatter_pass(
      x[:, :DH], x[:, DH:], src2d, dst2d, with_counts=True)
  h1 = _tc_layer(parts1, cnts, x, W1l, b1l, W1r)
  (parts2,) = _sc_scatter_pass(
      h1[:, :DH], h1[:, DH:], src2d, dst2d, with_counts=False)
  return _tc_head(parts2, cnts, h1, W2l, b2l, W2r, Wlin1, blin1, Wout, bout)


# speedup vs baseline: 1.1062x; 1.0099x over previous
import dataclasses
import functools

import jax
import jax.numpy as jnp
from jax import lax
from jax.experimental import pallas as pl
from jax.experimental.pallas import tpu as pltpu
from jax.experimental.pallas import tpu_sc as plsc

N_NODES = 10000
D = 128
DH = 64
NC = 2
NS = 16
NW = NC * NS
L = 16
CHUNK = 128
C0_CHUNKS_P1, C1_CHUNKS_P1 = 124, 34
C0_CHUNKS_P2, C1_CHUNKS_P2 = 120, 38
PAD_CHUNKS = max(
    NS * C0_CHUNKS_P1 + (NS - 1) * C1_CHUNKS_P1 + C0_CHUNKS_P1,
    NS * C0_CHUNKS_P2 + (NS - 1) * C1_CHUNKS_P2 + C0_CHUNKS_P2)
NPAD = 10112
ROWS_PER_TILE = NPAD // NS


def _sc_scatter_pass(feat_l, feat_r, src2d, dst2d, with_counts, c0, c1):
  mesh = plsc.VectorSubcoreMesh(core_axis_name="c", subcore_axis_name="s")
  out_types = [jax.ShapeDtypeStruct((NC, NPAD, D), jnp.float32)]
  scratch = [
      pltpu.VMEM((c0, CHUNK), jnp.int32),
      pltpu.VMEM((c0, CHUNK), jnp.int32),
      pltpu.VMEM((CHUNK, DH), jnp.float32),
      pltpu.VMEM((CHUNK, DH), jnp.float32),
      pltpu.VMEM((CHUNK, DH), jnp.float32),
      pltpu.VMEM_SHARED((NPAD, DH), jnp.float32),
      pltpu.SemaphoreType.DMA,
      pltpu.SemaphoreType.DMA,
  ]
  if with_counts:
    out_types.append(jax.ShapeDtypeStruct((NW, NPAD), jnp.float32))
    scratch.append(pltpu.VMEM((NPAD,), jnp.float32))

  cp = pltpu.CompilerParams()
  if "needs_layout_passes" in pltpu.CompilerParams.__dataclass_fields__:
    cp = dataclasses.replace(cp, needs_layout_passes=False)
  if "use_tc_tiling_on_sc" in pltpu.CompilerParams.__dataclass_fields__:
    cp = dataclasses.replace(cp, use_tc_tiling_on_sc=False)

  @functools.partial(
      pl.kernel, out_type=tuple(out_types), mesh=mesh, scratch_types=scratch,
      compiler_params=cp)
  def k(fl_hbm, fr_hbm, src_hbm, dst_hbm, out_hbm, *rest):
    if with_counts:
      cnt_hbm, srcv, dstv, buf_a, buf_b, zbuf, acc, sem_a, sem_b, cntv = rest
    else:
      srcv, dstv, buf_a, buf_b, zbuf, acc, sem_a, sem_b = rest
    c = lax.axis_index("c")
    s = lax.axis_index("s")
    wid = c * NS + s
    rows = pl.ds(s * ROWS_PER_TILE, ROWS_PER_TILE)
    off = jnp.where(c == 0, s * c0, NS * c0 + s * c1)
    nck = jnp.where(c == 0, c0, c1)

    pltpu.sync_copy(src_hbm.at[pl.ds(off, c0)], srcv)
    pltpu.sync_copy(dst_hbm.at[pl.ds(off, c0)], dstv)

    zero16 = jnp.zeros((L,), jnp.float32)

    @pl.loop(0, CHUNK)
    def _(r):
      for q in range(DH // L):
        zbuf[r, pl.ds(q * L, L)] = zero16

    if with_counts:
      @pl.loop(0, NPAD // L)
      def _(i):
        cntv[pl.ds(i * L, L)] = zero16

    ones16 = jnp.ones((L,), jnp.float32)

    for half, feat_hbm in enumerate((fl_hbm, fr_hbm)):
      base = s * ROWS_PER_TILE
      for kk in range(ROWS_PER_TILE // CHUNK):
        pltpu.sync_copy(zbuf, acc.at[pl.ds(base + kk * CHUNK, CHUNK)])
      rem = ROWS_PER_TILE % CHUNK
      if rem:
        pltpu.sync_copy(
            zbuf.at[pl.ds(0, rem)],
            acc.at[pl.ds(base + ROWS_PER_TILE - rem, rem)])
      plsc.subcore_barrier()

      pltpu.async_copy(feat_hbm.at[srcv.at[0]], buf_a, sem_a)

      do_counts = with_counts and half == 0

      @pl.loop(0, nck, step=2)
      def _(j):
        pltpu.async_copy(feat_hbm.at[srcv.at[j + 1]], buf_b, sem_b)
        if do_counts:
          for jo in (0, 1):
            for q in range(CHUNK // L):
              idx = dstv[j + jo, pl.ds(q * L, L)]
              plsc.addupdate_scatter(cntv, [idx], ones16)
        pltpu.make_async_copy(feat_hbm.at[srcv.at[j]], buf_a, sem_a).wait()
        pltpu.sync_copy(buf_a, acc.at[dstv.at[j]], add=True)

        @pl.when(j + 2 < nck)
        def _():
          pltpu.async_copy(feat_hbm.at[srcv.at[j + 2]], buf_a, sem_a)

        pltpu.make_async_copy(feat_hbm.at[srcv.at[j + 1]], buf_b, sem_b).wait()
        pltpu.sync_copy(buf_b, acc.at[dstv.at[j + 1]], add=True)

      if do_counts:
        pltpu.sync_copy(cntv, cnt_hbm.at[wid])
      plsc.subcore_barrier()

      pltpu.sync_copy(acc.at[rows],
                      out_hbm.at[c].at[rows, pl.ds(half * DH, DH)])
      plsc.subcore_barrier()

  return k(feat_l, feat_r, src2d, dst2d)


def _dot_t(a, w):
  return lax.dot_general(a, w, (((1,), (1,)), ((), ())),
                         preferred_element_type=jnp.float32)


def _combine(p_ref, c_ref):
  agg = (p_ref[0] + p_ref[1])[:N_NODES]
  cnt = jnp.sum(c_ref[...], axis=0)[:N_NODES]
  return agg / jnp.maximum(cnt, 1.0)[:, None]


def _tc_layer(parts, cnts, h_in, wl, bl, wr):
  def body(p_ref, c_ref, x_ref, wl_ref, bl_ref, wr_ref, o_ref):
    agg = _combine(p_ref, c_ref)
    h = (_dot_t(agg, wl_ref[...]) + bl_ref[...][None, :]
         + _dot_t(x_ref[...], wr_ref[...]))
    o_ref[...] = jnp.maximum(h, 0.0)

  return pl.pallas_call(
      body,
      out_shape=jax.ShapeDtypeStruct((N_NODES, D), jnp.float32),
  )(parts, cnts, h_in, wl, bl, wr)


def _tc_head(parts, cnts, h1, w2l, b2l, w2r, wlin1, blin1, wout, bout):
  def body(p_ref, c_ref, h1_ref, w2l_ref, b2l_ref, w2r_ref, wlin1_ref,
           blin1_ref, wout_ref, bout_ref, o_ref):
    agg = _combine(p_ref, c_ref)
    h2 = (_dot_t(agg, w2l_ref[...]) + b2l_ref[...][None, :]
          + _dot_t(h1_ref[...], w2r_ref[...]))
    h2 = jnp.maximum(h2, 0.0)
    g = jnp.mean(h2, axis=0, keepdims=True)
    z = _dot_t(g, wlin1_ref[...]) + blin1_ref[...][None, :]
    z = jnp.maximum(z, 0.0)
    ssum = jnp.sum(z * wout_ref[...]) + bout_ref[0]
    sv = jax.nn.sigmoid(ssum)
    o_ref[...] = jnp.full((N_NODES, 1), sv, jnp.float32)

  return pl.pallas_call(
      body,
      out_shape=jax.ShapeDtypeStruct((N_NODES, 1), jnp.float32),
  )(parts, cnts, h1, w2l, b2l, w2r, wlin1, blin1, wout, bout)


@jax.jit
def kernel(x, edge_index, W1l, b1l, W1r, W2l, b2l, W2r, Wlin1, blin1, Wout,
           bout):
  e = edge_index.astype(jnp.int32)
  src, dst = e[0], e[1]
  epad = PAD_CHUNKS * CHUNK
  n_extra = epad - src.shape[0]
  pad_dst = N_NODES + jnp.arange(n_extra, dtype=jnp.int32) % (NPAD - N_NODES)
  src2d = jnp.concatenate(
      [src, jnp.zeros((n_extra,), jnp.int32)]).reshape(PAD_CHUNKS, CHUNK)
  dst2d = jnp.concatenate([dst, pad_dst]).reshape(PAD_CHUNKS, CHUNK)

  parts1, cnts = _sc_scatter_pass(
      x[:, :DH], x[:, DH:], src2d, dst2d, with_counts=True,
      c0=C0_CHUNKS_P1, c1=C1_CHUNKS_P1)
  h1 = _tc_layer(parts1, cnts, x, W1l, b1l, W1r)
  (parts2,) = _sc_scatter_pass(
      h1[:, :DH], h1[:, DH:], src2d, dst2d, with_counts=False,
      c0=C0_CHUNKS_P2, c1=C1_CHUNKS_P2)
  return _tc_head(parts2, cnts, h1, W2l, b2l, W2r, Wlin1, blin1, Wout, bout)
